# baseline (speedup 1.0000x reference)
import functools

import jax
import jax.numpy as jnp
from jax import lax
from jax.experimental import pallas as pl
from jax.experimental.pallas import tpu as pltpu
from jax.experimental.pallas import tpu_sc as plsc

E = 1_000_000
EP = 1_000_064
DIM = 64
BATCH, SEQ = 4096, 200
B = BATCH * SEQ
TCOLS = EP // 128
PROWS = EP // 2

NC, NS = 2, 16
NW = NC * NS
B_PER_W = B // NW
IDXW = 128
CHUNK = 256
SUB = CHUNK // IDXW
NCHUNK = B_PER_W // CHUNK
NBUF = 4
IDX_ROWS = B_PER_W // IDXW


def _repack_body(wt_ref, out_ref):
    x = wt_ref[...]
    t = x.T
    r = lax.broadcasted_iota(jnp.int32, (64, 128), 0)
    k = lax.broadcasted_iota(jnp.int32, (64, 128), 1)
    e0 = (k == 2 * r).astype(jnp.float32)
    e1 = (k == 2 * r + 1).astype(jnp.float32)
    y0 = jax.lax.dot(e0, t, preferred_element_type=jnp.float32)
    y1 = jax.lax.dot(e1, t, preferred_element_type=jnp.float32)
    out_ref[...] = jnp.concatenate([y0, y1], axis=1)


_repack = pl.pallas_call(
    _repack_body,
    grid=(TCOLS,),
    in_specs=[pl.BlockSpec((64, 128), lambda j: (0, j))],
    out_specs=pl.BlockSpec((64, 128), lambda j: (j, 0)),
    out_shape=jax.ShapeDtypeStruct((PROWS, 128), jnp.float32),
)


_mesh = plsc.VectorSubcoreMesh(core_axis_name="c", subcore_axis_name="s")


@functools.partial(
    pl.kernel,
    mesh=_mesh,
    compiler_params=pltpu.CompilerParams(use_tc_tiling_on_sc=False),
    out_type=jax.ShapeDtypeStruct((B, DIM), jnp.float32),
    scratch_types=[
        pltpu.VMEM((IDX_ROWS, IDXW), jnp.int32),
        pltpu.VMEM((NBUF, CHUNK, DIM), jnp.float32),
        pltpu.SemaphoreType.DMA((NBUF,)),
        pltpu.SemaphoreType.DMA((NBUF,)),
    ],
)
def _gather(idx_hbm, table_hbm, out_hbm, idx_v, rows_v, gsem, ssem):
    wid = lax.axis_index("s") * NC + lax.axis_index("c")
    base = wid * B_PER_W

    idx_row0 = pl.multiple_of(wid * IDX_ROWS, 8)
    pltpu.sync_copy(idx_hbm.at[pl.ds(idx_row0, IDX_ROWS)], idx_v)

    def start_gather(c, b):
        for j in range(SUB):
            pltpu.async_copy(
                table_hbm.at[idx_v.at[c * SUB + j]],
                rows_v.at[b, pl.ds(j * IDXW, IDXW)],
                gsem.at[b],
            )

    def wait_gather(b):
        for j in range(SUB):
            pltpu.make_async_copy(
                table_hbm.at[idx_v.at[j]],
                rows_v.at[b, pl.ds(j * IDXW, IDXW)],
                gsem.at[b],
            ).wait()

    def start_store(c, b):
        off = pl.multiple_of(base + c * CHUNK, CHUNK)
        pltpu.async_copy(rows_v.at[b], out_hbm.at[pl.ds(off, CHUNK)], ssem.at[b])

    def wait_store(b):
        pltpu.make_async_copy(
            rows_v.at[b], out_hbm.at[pl.ds(base, CHUNK)], ssem.at[b]
        ).wait()

    for b in range(NBUF):
        start_gather(b, b)

    @pl.loop(0, NCHUNK, step=NBUF)
    def _ring(g):
        for b in range(NBUF):
            c = g + b
            wait_gather(b)
            start_store(c, b)
            nxt = c + NBUF

            @pl.when(nxt < NCHUNK)
            def _():
                wait_store(b)
                start_gather(nxt, b)

    for b in range(NBUF):
        wait_store(b)


def _otrans_body(g_ref, out_ref):
    for si in range(8):
        out_ref[si, :, :] = g_ref[:, si, :].T


_otrans = pl.pallas_call(
    _otrans_body,
    grid=(SEQ // 8, BATCH // 128),
    in_specs=[pl.BlockSpec((128, 8, DIM), lambda s, b: (b, s, 0))],
    out_specs=pl.BlockSpec((8, DIM, 128), lambda s, b: (s, 0, b)),
    out_shape=jax.ShapeDtypeStruct((SEQ, DIM, BATCH), jnp.float32),
)


def kernel(x, weight):
    idx = x.reshape(B // IDXW, IDXW).astype(jnp.int32)
    wpack = _repack(weight.T)
    wlin = wpack.reshape(EP, DIM)
    g = _gather(idx, wlin)
    out_t = _otrans(g.reshape(BATCH, SEQ, DIM))
    return jnp.transpose(out_t, (2, 0, 1))


# speedup vs baseline: 4.3257x; 4.3257x over previous
import functools

import jax
import jax.numpy as jnp
from jax import lax
from jax.experimental import pallas as pl
from jax.experimental.pallas import tpu as pltpu
from jax.experimental.pallas import tpu_sc as plsc

E = 1_000_000
EP = 1_000_064
DIM = 64
BATCH, SEQ = 4096, 200
B = BATCH * SEQ
PROWS = EP // 2
RW = 1664
RBLK = 601

NC, NS = 2, 16
NW = NC * NS
B_PER_W = B // NW
IDXW = 128
CHUNK = 256
SUB = CHUNK // IDXW
NCHUNK = B_PER_W // CHUNK
NBUF = 4
IDX_ROWS = B_PER_W // IDXW


def _repack_body(wt_ref, out_ref):
    x = wt_ref[...]
    r = lax.broadcasted_iota(jnp.int32, (64, 128), 0)
    k = lax.broadcasted_iota(jnp.int32, (64, 128), 1)
    e0 = (k == 2 * r).astype(jnp.float32)
    e1 = (k == 2 * r + 1).astype(jnp.float32)
    for j in range(RW // 128):
        t = x[:, j * 128:(j + 1) * 128].T
        y0 = jax.lax.dot(e0, t, preferred_element_type=jnp.float32)
        y1 = jax.lax.dot(e1, t, preferred_element_type=jnp.float32)
        out_ref[j * 64:(j + 1) * 64, :] = jnp.concatenate([y0, y1], axis=1)


_repack = pl.pallas_call(
    _repack_body,
    grid=(RBLK,),
    in_specs=[pl.BlockSpec((DIM, RW), lambda j: (0, j))],
    out_specs=pl.BlockSpec((RW // 2, 128), lambda j: (j, 0)),
    out_shape=jax.ShapeDtypeStruct((PROWS, 128), jnp.float32),
)


_mesh = plsc.VectorSubcoreMesh(core_axis_name="c", subcore_axis_name="s")


@functools.partial(
    pl.kernel,
    mesh=_mesh,
    compiler_params=pltpu.CompilerParams(use_tc_tiling_on_sc=False),
    out_type=jax.ShapeDtypeStruct((B, DIM), jnp.float32),
    scratch_types=[
        pltpu.VMEM((IDX_ROWS, IDXW), jnp.int32),
        pltpu.VMEM((NBUF, CHUNK, DIM), jnp.float32),
        pltpu.SemaphoreType.DMA((NBUF,)),
        pltpu.SemaphoreType.DMA((NBUF,)),
    ],
)
def _gather(idx_hbm, table_hbm, out_hbm, idx_v, rows_v, gsem, ssem):
    wid = lax.axis_index("s") * NC + lax.axis_index("c")
    base = wid * B_PER_W

    idx_row0 = pl.multiple_of(wid * IDX_ROWS, 8)
    pltpu.sync_copy(idx_hbm.at[pl.ds(idx_row0, IDX_ROWS)], idx_v)

    def start_gather(c, b):
        for j in range(SUB):
            pltpu.async_copy(
                table_hbm.at[idx_v.at[c * SUB + j]],
                rows_v.at[b, pl.ds(j * IDXW, IDXW)],
                gsem.at[b],
            )

    def wait_gather(b):
        for j in range(SUB):
            pltpu.make_async_copy(
                table_hbm.at[idx_v.at[j]],
                rows_v.at[b, pl.ds(j * IDXW, IDXW)],
                gsem.at[b],
            ).wait()

    def start_store(c, b):
        off = pl.multiple_of(base + c * CHUNK, CHUNK)
        pltpu.async_copy(rows_v.at[b], out_hbm.at[pl.ds(off, CHUNK)], ssem.at[b])

    def wait_store(b):
        pltpu.make_async_copy(
            rows_v.at[b], out_hbm.at[pl.ds(base, CHUNK)], ssem.at[b]
        ).wait()

    for b in range(NBUF):
        start_gather(b, b)

    @pl.loop(0, NCHUNK, step=NBUF)
    def _ring(g):
        for b in range(NBUF):
            c = g + b
            wait_gather(b)
            start_store(c, b)
            nxt = c + NBUF

            @pl.when(nxt < NCHUNK)
            def _():
                wait_store(b)
                start_gather(nxt, b)

    for b in range(NBUF):
        wait_store(b)


def _otrans_body(g_ref, out_ref):
    m = lax.broadcasted_iota(jnp.int32, (64, 128), 0)
    c = lax.broadcasted_iota(jnp.int32, (64, 128), 1)
    b0 = (c == 2 * m).astype(jnp.float32)
    b1 = (c == 2 * m + 1).astype(jnp.float32)
    for si in range(8):
        t = g_ref[si].T
        h0 = t[0:64, :]
        h1 = t[64:128, :]
        out_ref[si] = (
            jax.lax.dot(h0, b0, preferred_element_type=jnp.float32)
            + jax.lax.dot(h1, b1, preferred_element_type=jnp.float32)
        )


_otrans = pl.pallas_call(
    _otrans_body,
    grid=(SEQ // 8, BATCH // 128),
    in_specs=[pl.BlockSpec((8, DIM, 128), lambda s, b: (s, b, 0))],
    out_specs=pl.BlockSpec((8, DIM, 128), lambda s, b: (s, 0, b)),
    out_shape=jax.ShapeDtypeStruct((SEQ, DIM, BATCH), jnp.float32),
)


def kernel(x, weight):
    idx = x.T.reshape(-1).reshape(B // IDXW, IDXW).astype(jnp.int32)
    wpack = _repack(weight.T)
    wlin = wpack.reshape(EP, DIM)
    g = _gather(idx, wlin)
    gp = g.reshape(SEQ, BATCH // 2, 128)
    out_t = _otrans(gp)
    return jnp.transpose(out_t, (2, 0, 1))


# speedup vs baseline: 6.1823x; 1.4292x over previous
import functools

import jax
import jax.numpy as jnp
from jax import lax
from jax.experimental import pallas as pl
from jax.experimental.pallas import tpu as pltpu
from jax.experimental.pallas import tpu_sc as plsc

E = 1_000_000
EP = 1_000_064
DIM = 64
BATCH, SEQ = 4096, 200
B = BATCH * SEQ
PROWS = EP // 2
RW = 1664
RBLK = 601

NC, NS = 2, 16
NW = NC * NS
B_PER_W = B // NW
IDXW = 128
CHUNK = 256
SUB = CHUNK // IDXW
NCHUNK = B_PER_W // CHUNK
NBUF = 4
IDX_ROWS = B_PER_W // IDXW


_NT = (((1,), (1,)), ((), ()))


def _repack_body(wt_ref, out_ref):
    x = wt_ref[...]
    r = lax.broadcasted_iota(jnp.int32, (64, 128), 0)
    k = lax.broadcasted_iota(jnp.int32, (64, 128), 1)
    e0 = (k == 2 * r).astype(jnp.float32)
    e1 = (k == 2 * r + 1).astype(jnp.float32)
    for j in range(RW // 128):
        xj = x[:, j * 128:(j + 1) * 128]
        y0 = jax.lax.dot_general(e0, xj, _NT, preferred_element_type=jnp.float32)
        y1 = jax.lax.dot_general(e1, xj, _NT, preferred_element_type=jnp.float32)
        out_ref[j * 64:(j + 1) * 64, :] = jnp.concatenate([y0, y1], axis=1)


_repack = pl.pallas_call(
    _repack_body,
    grid=(RBLK,),
    in_specs=[pl.BlockSpec((DIM, RW), lambda j: (0, j))],
    out_specs=pl.BlockSpec((RW // 2, 128), lambda j: (j, 0)),
    out_shape=jax.ShapeDtypeStruct((PROWS, 128), jnp.float32),
)


_mesh = plsc.VectorSubcoreMesh(core_axis_name="c", subcore_axis_name="s")


@functools.partial(
    pl.kernel,
    mesh=_mesh,
    compiler_params=pltpu.CompilerParams(use_tc_tiling_on_sc=False),
    out_type=jax.ShapeDtypeStruct((B, DIM), jnp.float32),
    scratch_types=[
        pltpu.VMEM((IDX_ROWS, IDXW), jnp.int32),
        pltpu.VMEM((NBUF, CHUNK, DIM), jnp.float32),
        pltpu.SemaphoreType.DMA((NBUF,)),
        pltpu.SemaphoreType.DMA((NBUF,)),
    ],
)
def _gather(idx_hbm, table_hbm, out_hbm, idx_v, rows_v, gsem, ssem):
    wid = lax.axis_index("s") * NC + lax.axis_index("c")
    base = wid * B_PER_W

    idx_row0 = pl.multiple_of(wid * IDX_ROWS, 8)
    pltpu.sync_copy(idx_hbm.at[pl.ds(idx_row0, IDX_ROWS)], idx_v)

    def start_gather(c, b):
        for j in range(SUB):
            pltpu.async_copy(
                table_hbm.at[idx_v.at[c * SUB + j]],
                rows_v.at[b, pl.ds(j * IDXW, IDXW)],
                gsem.at[b],
            )

    def wait_gather(b):
        for j in range(SUB):
            pltpu.make_async_copy(
                table_hbm.at[idx_v.at[j]],
                rows_v.at[b, pl.ds(j * IDXW, IDXW)],
                gsem.at[b],
            ).wait()

    def start_store(c, b):
        off = pl.multiple_of(base + c * CHUNK, CHUNK)
        pltpu.async_copy(rows_v.at[b], out_hbm.at[pl.ds(off, CHUNK)], ssem.at[b])

    def wait_store(b):
        pltpu.make_async_copy(
            rows_v.at[b], out_hbm.at[pl.ds(base, CHUNK)], ssem.at[b]
        ).wait()

    for b in range(NBUF):
        start_gather(b, b)

    @pl.loop(0, NCHUNK, step=NBUF)
    def _ring(g):
        for b in range(NBUF):
            c = g + b
            wait_gather(b)
            start_store(c, b)
            nxt = c + NBUF

            @pl.when(nxt < NCHUNK)
            def _():
                wait_store(b)
                start_gather(nxt, b)

    for b in range(NBUF):
        wait_store(b)


_TN = (((0,), (0,)), ((), ()))


def _otrans_body(g_ref, out_ref):
    m = lax.broadcasted_iota(jnp.int32, (64, 128), 0)
    c = lax.broadcasted_iota(jnp.int32, (64, 128), 1)
    b0 = (c == 2 * m).astype(jnp.float32)
    b1 = (c == 2 * m + 1).astype(jnp.float32)
    for si in range(8):
        for gq in range(8):
            xs = g_ref[si, gq * 64:(gq + 1) * 64, :]
            xs0 = xs[:, 0:DIM]
            xs1 = xs[:, DIM:128]
            out_ref[si, :, gq * 128:(gq + 1) * 128] = (
                jax.lax.dot_general(xs0, b0, _TN, preferred_element_type=jnp.float32)
                + jax.lax.dot_general(xs1, b1, _TN, preferred_element_type=jnp.float32)
            )


_otrans = pl.pallas_call(
    _otrans_body,
    grid=(SEQ // 8, BATCH // 1024),
    in_specs=[pl.BlockSpec((8, 512, 128), lambda s, b: (s, b, 0))],
    out_specs=pl.BlockSpec((8, DIM, 1024), lambda s, b: (s, 0, b)),
    out_shape=jax.ShapeDtypeStruct((SEQ, DIM, BATCH), jnp.float32),
)


def kernel(x, weight):
    idx = x.T.reshape(-1).reshape(B // IDXW, IDXW).astype(jnp.int32)
    wpack = _repack(weight.T)
    wlin = wpack.reshape(EP, DIM)
    g = _gather(idx, wlin)
    gp = g.reshape(SEQ, BATCH // 2, 128)
    out_t = _otrans(gp)
    return jnp.transpose(out_t, (2, 0, 1))
